# Initial kernel scaffold; baseline (speedup 1.0000x reference)
import jax
import jax.numpy as jnp
from jax.experimental import pallas as pl


def kernel(atom_tables, bond_aa_tables, motif_table, am_table, mm_table, ma_table, Wa1, ba1, Wa2, ba2, Wm1, bm1, Wm2, bm2, x_atom, x_motif, edge_index_aa, edge_attr_aa, edge_index_am, edge_attr_am, edge_index_mm, edge_attr_mm, edge_index_ma, edge_attr_ma, batch_atom, batch_motif):
    raise NotImplementedError("write your pallas kernel here")


# speedup vs baseline: 4.1996x; 4.1996x over previous
import functools

import jax
import jax.numpy as jnp
from jax import lax
from jax.experimental import pallas as pl
from jax.experimental.pallas import tpu as pltpu
from jax.experimental.pallas import tpu_sc as plsc

F32 = jnp.float32
HIGHEST = lax.Precision.HIGHEST

N_A = 10000
N_M = 2000
D = 256
L = 3
B = 64

CHUNK = 128
EDGE_ALIGN = 16 * CHUNK
ACC_A = 10048
ACC_M = 2048
ZROWS = 624


def _ceil_to(x, m):
    return ((x + m - 1) // m) * m


def _encode_atoms(x_atom, Da, base_a):
    Bn = 2000
    nb = N_A // Bn

    def body(x_ref, da_ref, base_ref, o_ref):
        xf = x_ref[...].astype(F32)
        acc = jnp.broadcast_to(base_ref[...], (Bn, D))
        for i in range(9):
            acc = acc + xf[:, i:i + 1] * da_ref[i:i + 1, :]
        o_ref[...] = acc

    return pl.pallas_call(
        body,
        grid=(nb,),
        in_specs=[
            pl.BlockSpec((Bn, 9), lambda j: (j, 0)),
            pl.BlockSpec((9, D), lambda j: (0, 0)),
            pl.BlockSpec((1, D), lambda j: (0, 0)),
        ],
        out_specs=pl.BlockSpec((Bn, D), lambda j: (j, 0)),
        out_shape=jax.ShapeDtypeStruct((N_A, D), F32),
    )(x_atom, Da, base_a)


def _encode_motifs(x_motif, motif_table_pad):
    def body(ids_ref, tab_ref, o_ref):
        ids = ids_ref[...]
        iota = lax.broadcasted_iota(jnp.int32, (1, 64), 1)
        oh = (ids == iota).astype(F32)
        o_ref[...] = jnp.dot(oh, tab_ref[...], precision=HIGHEST)

    return pl.pallas_call(
        body,
        in_specs=[
            pl.BlockSpec((N_M, 1), lambda: (0, 0)),
            pl.BlockSpec((64, D), lambda: (0, 0)),
        ],
        out_specs=pl.BlockSpec((N_M, D), lambda: (0, 0)),
        out_shape=jax.ShapeDtypeStruct((N_M, D), F32),
    )(x_motif, motif_table_pad)


def _build_table(h, lut, bv):
    n = h.shape[0]
    c = lut.shape[0]
    nb = n // bv

    def body(h_ref, lut_ref, o_ref):
        t = jnp.maximum(h_ref[...][:, None, :] + lut_ref[...][None, :, :], 0.0)
        o_ref[...] = t.reshape(bv * c, 128)

    return pl.pallas_call(
        body,
        grid=(2, nb),
        in_specs=[
            pl.BlockSpec((bv, 128), lambda hf, j: (j, hf)),
            pl.BlockSpec((c, 128), lambda hf, j: (0, hf)),
        ],
        out_specs=pl.BlockSpec((bv * c, 128), lambda hf, j: (hf * nb + j, 0)),
        out_shape=jax.ShapeDtypeStruct((2 * n * c, 128), F32),
    )(h, lut)


def _mlp(h, agg, w1, b1, w2, b2):
    n = h.shape[0]
    bn = 2000
    nb = n // bn

    def body(h_ref, a0_ref, a1_ref, w1_ref, b1_ref, w2_ref, b2_ref, o_ref):
        x = h_ref[...] + jnp.concatenate([a0_ref[...], a1_ref[...]], axis=1)
        y = jnp.maximum(jnp.dot(x, w1_ref[...], precision=HIGHEST) + b1_ref[...], 0.0)
        o_ref[...] = jnp.maximum(
            jnp.dot(y, w2_ref[...], precision=HIGHEST) + b2_ref[...], 0.0)

    return pl.pallas_call(
        body,
        grid=(nb,),
        in_specs=[
            pl.BlockSpec((bn, D), lambda j: (j, 0)),
            pl.BlockSpec((bn, 128), lambda j: (j, 0)),
            pl.BlockSpec((bn, 128), lambda j: (nb + j, 0)),
            pl.BlockSpec((D, D), lambda j: (0, 0)),
            pl.BlockSpec((1, D), lambda j: (0, 0)),
            pl.BlockSpec((D, D), lambda j: (0, 0)),
            pl.BlockSpec((1, D), lambda j: (0, 0)),
        ],
        out_specs=pl.BlockSpec((bn, D), lambda j: (j, 0)),
        out_shape=jax.ShapeDtypeStruct((n, D), F32),
    )(h, agg, agg, w1, b1, w2, b2)


def _pool(xs, batch3d):
    n = xs[0].shape[0]
    bn = 2000
    nb = n // bn

    def body(b_ref, x0, x1, x2, x3, o_ref):
        j = pl.program_id(0)
        ids = b_ref[0, 0, :]
        iota = lax.broadcasted_iota(jnp.int32, (B, bn), 0)
        oh = (iota == ids[None, :]).astype(F32)
        xcat = jnp.concatenate([x0[...], x1[...], x2[...], x3[...]], axis=1)
        part = jnp.dot(oh, xcat, precision=HIGHEST)

        @pl.when(j == 0)
        def _():
            o_ref[...] = jnp.zeros_like(o_ref)

        o_ref[...] += part

    xspec = pl.BlockSpec((bn, D), lambda j: (j, 0))
    return pl.pallas_call(
        body,
        grid=(nb,),
        in_specs=[pl.BlockSpec((1, 1, bn), lambda j: (j, 0, 0))] + [xspec] * 4,
        out_specs=pl.BlockSpec((B, 4 * D), lambda j: (0, 0)),
        out_shape=jax.ShapeDtypeStruct((B, 4 * D), F32),
    )(batch3d, *xs)


PAD_AA = _ceil_to(160000, EDGE_ALIGN)
PAD_MA = _ceil_to(20000, EDGE_ALIGN)
PAD_MM = _ceil_to(8000, EDGE_ALIGN)
PAD_AM = _ceil_to(20000, EDGE_ALIGN)

@functools.lru_cache(maxsize=1)
def _get_sc_aggregate():
    mesh = plsc.VectorSubcoreMesh(core_axis_name="c", subcore_axis_name="s")

    @functools.partial(
        pl.kernel,
        mesh=mesh,
        out_type=[
            jax.ShapeDtypeStruct((2 * N_A, 128), F32),
            jax.ShapeDtypeStruct((2 * N_M, 128), F32),
        ],
        scratch_types=[
            pltpu.VMEM((CHUNK,), jnp.int32),
            pltpu.VMEM((CHUNK,), jnp.int32),
            pltpu.VMEM((CHUNK, 128), F32),
            pltpu.VMEM_SHARED((ACC_A, 128), F32),
            pltpu.VMEM_SHARED((ACC_M, 128), F32),
            pltpu.SemaphoreType.DMA,
        ],
    )
    def _sc_aggregate(taa, tma, tmm, tam, gaa, daa, gma, dma, gmm, dmm,
                      gam, dam, zrows, out_a, out_m,
                      idx_v, dst_v, rows_v, acc_a, acc_m, sem):
        _sc_body(taa, tma, tmm, tam, gaa, daa, gma, dma, gmm, dmm, gam, dam,
                 zrows, out_a, out_m, idx_v, dst_v, rows_v, acc_a, acc_m, sem)

    return _sc_aggregate


def _m8(x):
    return pl.multiple_of(x, 8)


def _sc_body(taa, tma, tmm, tam, gaa, daa, gma, dma, gmm, dmm, gam, dam,
             zrows, out_a, out_m, idx_v, dst_v, rows_v, acc_a, acc_m, sem):
    cid = lax.axis_index("c")
    sid = lax.axis_index("s")

    pltpu.sync_copy(zrows, acc_a.at[pl.ds(_m8(sid * 624), 624)])
    pltpu.sync_copy(zrows.at[pl.ds(0, 128)],
                    acc_m.at[pl.ds(_m8(sid * 128), 128)])

    @pl.when(sid == 15)
    def _():
        pltpu.sync_copy(zrows.at[pl.ds(0, ACC_A - 16 * 624)],
                        acc_a.at[pl.ds(16 * 624, ACC_A - 16 * 624)])

    plsc.subcore_barrier()

    def process(tab, gcat, gd, epad, acc):
        per = epad // 16
        nch = per // CHUNK

        def step(i, carry):
            off = _m8(sid * per + i * CHUNK)
            pltpu.sync_copy(gcat.at[pl.ds(_m8(cid * epad + off), CHUNK)], idx_v)
            pltpu.sync_copy(gd.at[pl.ds(off, CHUNK)], dst_v)
            pltpu.async_copy(tab.at[idx_v], rows_v, sem).wait()
            pltpu.sync_copy(rows_v, acc.at[dst_v], add=True)
            return carry

        lax.fori_loop(0, nch, step, 0)

    process(taa, gaa, daa, PAD_AA, acc_a)
    process(tma, gma, dma, PAD_MA, acc_a)
    process(tmm, gmm, dmm, PAD_MM, acc_m)
    process(tam, gam, dam, PAD_AM, acc_m)
    plsc.subcore_barrier()

    pltpu.sync_copy(acc_a.at[pl.ds(_m8(sid * 624), 624)],
                    out_a.at[pl.ds(_m8(cid * N_A + sid * 624), 624)])
    pltpu.sync_copy(acc_m.at[pl.ds(_m8(sid * 120), 120)],
                    out_m.at[pl.ds(_m8(cid * N_M + sid * 120), 120)])

    @pl.when(sid == 15)
    def _():
        pltpu.sync_copy(acc_a.at[pl.ds(16 * 624, N_A - 16 * 624)],
                        out_a.at[pl.ds(_m8(cid * N_A + 16 * 624),
                                       N_A - 16 * 624)])
        pltpu.sync_copy(acc_m.at[pl.ds(16 * 120, N_M - 16 * 120)],
                        out_m.at[pl.ds(_m8(cid * N_M + 16 * 120),
                                       N_M - 16 * 120)])


def _edge_arrays(src, dst, code, ncodes, nrows_half, epad, acc_rows, nreal):
    gidx = src * ncodes + code
    e = gidx.shape[0]
    pad = epad - e
    ar = jnp.arange(pad, dtype=jnp.int32)
    gidx = jnp.concatenate([gidx, (ar * 37) % nrows_half])
    dst = jnp.concatenate([dst, nreal + ar % (acc_rows - nreal)])
    gcat = jnp.concatenate([gidx, gidx + nrows_half])
    return gcat, dst


def kernel(atom_tables, bond_aa_tables, motif_table, am_table, mm_table, ma_table,
           Wa1, ba1, Wa2, ba2, Wm1, bm1, Wm2, bm2,
           x_atom, x_motif, edge_index_aa, edge_attr_aa, edge_index_am, edge_attr_am,
           edge_index_mm, edge_attr_mm, edge_index_ma, edge_attr_ma,
           batch_atom, batch_motif):
    Da = atom_tables[:, 1, :] - atom_tables[:, 0, :]
    base_a = jnp.sum(atom_tables[:, 0, :], axis=0)[None, :]
    bits = (jnp.arange(8, dtype=jnp.int32)[:, None]
            >> jnp.arange(3, dtype=jnp.int32)[None, :]) & 1
    lut_aa = (bond_aa_tables[0][bits[:, 0]]
              + bond_aa_tables[1][bits[:, 1]]
              + bond_aa_tables[2][bits[:, 2]])
    motif_table_pad = jnp.zeros((64, D), F32).at[:61].set(motif_table)

    code_aa = (edge_attr_aa[:, 0] + 2 * edge_attr_aa[:, 1]
               + 4 * edge_attr_aa[:, 2])
    gaa, daa = _edge_arrays(edge_index_aa[0], edge_index_aa[1], code_aa,
                            8, 8 * N_A, PAD_AA, ACC_A, N_A)
    gma, dma = _edge_arrays(edge_index_ma[0], edge_index_ma[1], edge_attr_ma,
                            2, 2 * N_M, PAD_MA, ACC_A, N_A)
    gmm, dmm = _edge_arrays(edge_index_mm[0], edge_index_mm[1], edge_attr_mm,
                            22, 22 * N_M, PAD_MM, ACC_M, N_M)
    gam, dam = _edge_arrays(edge_index_am[0], edge_index_am[1], edge_attr_am,
                            2, 2 * N_M, PAD_AM, ACC_M, N_M)
    zrows = jnp.zeros((ZROWS, 128), F32)

    h_a = _encode_atoms(x_atom, Da, base_a)
    h_m = _encode_motifs(x_motif, motif_table_pad)

    xs_a = [h_a]
    xs_m = [h_m]
    for l in range(L):
        taa = _build_table(h_a, lut_aa, 1000)
        tma = _build_table(h_m, ma_table, N_M)
        tmm = _build_table(h_m, mm_table, 200)
        tam = _build_table(h_a[:N_M], am_table, N_M)
        agg_a, agg_m = _get_sc_aggregate()(taa, tma, tmm, tam,
                                           gaa, daa, gma, dma, gmm, dmm,
                                           gam, dam, zrows)
        h_a = _mlp(h_a, agg_a, Wa1[l], ba1[l][None, :], Wa2[l], ba2[l][None, :])
        h_m = _mlp(h_m, agg_m, Wm1[l], bm1[l][None, :], Wm2[l], bm2[l][None, :])
        xs_a.append(h_a)
        xs_m.append(h_m)

    atom_embs = _pool(xs_a, jnp.reshape(batch_atom, (N_A // 2000, 1, 2000)))
    motif_embs = _pool(xs_m, jnp.reshape(batch_motif, (1, 1, 2000)))
    return jnp.concatenate([atom_embs, motif_embs], axis=1)


# speedup vs baseline: 5.3454x; 1.2728x over previous
import functools

import jax
import jax.numpy as jnp
from jax import lax
from jax.experimental import pallas as pl
from jax.experimental.pallas import tpu as pltpu
from jax.experimental.pallas import tpu_sc as plsc

F32 = jnp.float32
HIGHEST = lax.Precision.HIGHEST

N_A = 10000
N_M = 2000
D = 256
L = 3
B = 64

CHUNK = 128
G = 1
EDGE_ALIGN = 16 * CHUNK
ACC_A = 10048
ACC_M = 2048
ZROWS = 624


def _ceil_to(x, m):
    return ((x + m - 1) // m) * m


def _encode_atoms(x_atom, Da, base_a):
    Bn = 2000
    nb = N_A // Bn

    def body(x_ref, da_ref, base_ref, o_ref):
        xf = x_ref[...].astype(F32)
        acc = jnp.broadcast_to(base_ref[...], (Bn, D))
        for i in range(9):
            acc = acc + xf[:, i:i + 1] * da_ref[i:i + 1, :]
        o_ref[...] = acc

    return pl.pallas_call(
        body,
        grid=(nb,),
        in_specs=[
            pl.BlockSpec((Bn, 9), lambda j: (j, 0)),
            pl.BlockSpec((9, D), lambda j: (0, 0)),
            pl.BlockSpec((1, D), lambda j: (0, 0)),
        ],
        out_specs=pl.BlockSpec((Bn, D), lambda j: (j, 0)),
        out_shape=jax.ShapeDtypeStruct((N_A, D), F32),
    )(x_atom, Da, base_a)


def _encode_motifs(x_motif, motif_table_pad):
    def body(ids_ref, tab_ref, o_ref):
        ids = ids_ref[...]
        iota = lax.broadcasted_iota(jnp.int32, (1, 64), 1)
        oh = (ids == iota).astype(F32)
        o_ref[...] = jnp.dot(oh, tab_ref[...], precision=HIGHEST)

    return pl.pallas_call(
        body,
        in_specs=[
            pl.BlockSpec((N_M, 1), lambda: (0, 0)),
            pl.BlockSpec((64, D), lambda: (0, 0)),
        ],
        out_specs=pl.BlockSpec((N_M, D), lambda: (0, 0)),
        out_shape=jax.ShapeDtypeStruct((N_M, D), F32),
    )(x_motif, motif_table_pad)


def _build_table(h, lut, bv):
    n = h.shape[0]
    c = lut.shape[0]
    nb = n // bv

    def body(h_ref, lut_ref, o_ref):
        t = jnp.maximum(h_ref[...][:, None, :] + lut_ref[...][None, :, :], 0.0)
        o_ref[...] = t.reshape(bv * c, 128)

    return pl.pallas_call(
        body,
        grid=(2, nb),
        in_specs=[
            pl.BlockSpec((bv, 128), lambda hf, j: (j, hf)),
            pl.BlockSpec((c, 128), lambda hf, j: (0, hf)),
        ],
        out_specs=pl.BlockSpec((bv * c, 128), lambda hf, j: (hf * nb + j, 0)),
        out_shape=jax.ShapeDtypeStruct((2 * n * c, 128), F32),
    )(h, lut)


def _mlp(h, agg, w1, b1, w2, b2):
    n = h.shape[0]
    bn = 2000
    nb = n // bn

    def body(h_ref, a0_ref, a1_ref, w1_ref, b1_ref, w2_ref, b2_ref, o_ref):
        x = h_ref[...] + jnp.concatenate([a0_ref[...], a1_ref[...]], axis=1)
        y = jnp.maximum(jnp.dot(x, w1_ref[...], precision=HIGHEST) + b1_ref[...], 0.0)
        o_ref[...] = jnp.maximum(
            jnp.dot(y, w2_ref[...], precision=HIGHEST) + b2_ref[...], 0.0)

    return pl.pallas_call(
        body,
        grid=(nb,),
        in_specs=[
            pl.BlockSpec((bn, D), lambda j: (j, 0)),
            pl.BlockSpec((bn, 128), lambda j: (j, 0)),
            pl.BlockSpec((bn, 128), lambda j: (nb + j, 0)),
            pl.BlockSpec((D, D), lambda j: (0, 0)),
            pl.BlockSpec((1, D), lambda j: (0, 0)),
            pl.BlockSpec((D, D), lambda j: (0, 0)),
            pl.BlockSpec((1, D), lambda j: (0, 0)),
        ],
        out_specs=pl.BlockSpec((bn, D), lambda j: (j, 0)),
        out_shape=jax.ShapeDtypeStruct((n, D), F32),
    )(h, agg, agg, w1, b1, w2, b2)


def _pool(xs, batch3d):
    n = xs[0].shape[0]
    bn = 2000
    nb = n // bn

    def body(b_ref, x0, x1, x2, x3, o_ref):
        j = pl.program_id(0)
        ids = b_ref[0, 0, :]
        iota = lax.broadcasted_iota(jnp.int32, (B, bn), 0)
        oh = (iota == ids[None, :]).astype(F32)
        xcat = jnp.concatenate([x0[...], x1[...], x2[...], x3[...]], axis=1)
        part = jnp.dot(oh, xcat, precision=HIGHEST)

        @pl.when(j == 0)
        def _():
            o_ref[...] = jnp.zeros_like(o_ref)

        o_ref[...] += part

    xspec = pl.BlockSpec((bn, D), lambda j: (j, 0))
    return pl.pallas_call(
        body,
        grid=(nb,),
        in_specs=[pl.BlockSpec((1, 1, bn), lambda j: (j, 0, 0))] + [xspec] * 4,
        out_specs=pl.BlockSpec((B, 4 * D), lambda j: (0, 0)),
        out_shape=jax.ShapeDtypeStruct((B, 4 * D), F32),
    )(batch3d, *xs)


PAD_AA = _ceil_to(160000, EDGE_ALIGN)
PAD_MA = _ceil_to(20000, EDGE_ALIGN)
PAD_MM = _ceil_to(8000, EDGE_ALIGN)
PAD_AM = _ceil_to(20000, EDGE_ALIGN)

@functools.lru_cache(maxsize=1)
def _get_sc_aggregate():
    mesh = plsc.VectorSubcoreMesh(core_axis_name="c", subcore_axis_name="s")

    @functools.partial(
        pl.kernel,
        mesh=mesh,
        out_type=[
            jax.ShapeDtypeStruct((2 * N_A, 128), F32),
            jax.ShapeDtypeStruct((2 * N_M, 128), F32),
        ],
        scratch_types=[
            pltpu.VMEM((G * CHUNK,), jnp.int32),
            pltpu.VMEM((G * CHUNK,), jnp.int32),
            pltpu.VMEM((G, CHUNK), jnp.int32),
            pltpu.VMEM((G, CHUNK), jnp.int32),
            pltpu.VMEM((G * CHUNK, 128), F32),
            pltpu.VMEM((G * CHUNK, 128), F32),
            pltpu.VMEM_SHARED((ACC_A, 128), F32),
            pltpu.SemaphoreType.DMA,
            pltpu.SemaphoreType.DMA,
            pltpu.SemaphoreType.DMA,
            pltpu.SemaphoreType.DMA,
            pltpu.SemaphoreType.DMA,
            pltpu.SemaphoreType.DMA,
        ],
    )
    def _sc_aggregate(taa, tma, tmm, tam, gaa, daa, gma, dma, gmm, dmm,
                      gam, dam, zrows, out_a, out_m, *bufs):
        _sc_body(taa, tma, tmm, tam, gaa, daa, gma, dma, gmm, dmm, gam, dam,
                 zrows, out_a, out_m, bufs)

    return _sc_aggregate


def _m8(x):
    return pl.multiple_of(x, 8)


def _sc_body(taa, tma, tmm, tam, gaa, daa, gma, dma, gmm, dmm, gam, dam,
             zrows, out_a, out_m, bufs):
    (idx_a, idx_b, dst_a, dst_b, rows_a, rows_b, acc,
     is_a, is_b, gs_a, gs_b, ss_a, ss_b) = bufs
    cid = lax.axis_index("c")
    sid = lax.axis_index("s")

    pltpu.sync_copy(zrows, acc.at[pl.ds(_m8(sid * 624), 624)])

    @pl.when(sid == 15)
    def _():
        pltpu.sync_copy(zrows.at[pl.ds(0, ACC_A - 16 * 624)],
                        acc.at[pl.ds(16 * 624, ACC_A - 16 * 624)])

    plsc.subcore_barrier()

    def process(tab, gcat, gd, epad, acc):
        per = epad // 16
        nch = per // CHUNK
        npair = nch // (2 * G)
        base = sid * per

        def load(c0, idx_v, dst_v, isem):
            off = _m8(base + c0 * CHUNK)
            ds = [pltpu.async_copy(
                gcat.at[pl.ds(_m8(cid * epad + off), G * CHUNK)], idx_v, isem)]
            for g in range(G):
                ds.append(pltpu.async_copy(
                    gd.at[pl.ds(_m8(off + g * CHUNK), CHUNK)],
                    dst_v.at[g], isem))
            return ds

        def gathers(idx_v, rows_v, gsem):
            return [pltpu.async_copy(
                tab.at[idx_v.at[pl.ds(g * CHUNK, CHUNK)]],
                rows_v.at[pl.ds(g * CHUNK, CHUNK)], gsem) for g in range(G)]

        def scatters(rows_v, dst_v, ssem):
            return [pltpu.async_copy(
                rows_v.at[pl.ds(g * CHUNK, CHUNK)],
                acc.at[dst_v.at[g]], ssem, add=True) for g in range(G)]

        def pair(i, carry):
            la = load(i * 2 * G, idx_a, dst_a, is_a)
            lb = load(i * 2 * G + G, idx_b, dst_b, is_b)
            for d in la:
                d.wait()
            ga = gathers(idx_a, rows_a, gs_a)
            for d in lb:
                d.wait()
            gb = gathers(idx_b, rows_b, gs_b)
            for d in ga:
                d.wait()
            sa = scatters(rows_a, dst_a, ss_a)
            for d in gb:
                d.wait()
            sb = scatters(rows_b, dst_b, ss_b)
            for d in sa + sb:
                d.wait()
            return carry

        lax.fori_loop(0, npair, pair, 0)

        def rem_step(i, carry):
            c = npair * 2 * G + i
            off = _m8(base + c * CHUNK)
            pltpu.sync_copy(gcat.at[pl.ds(_m8(cid * epad + off), CHUNK)],
                            idx_a.at[pl.ds(0, CHUNK)])
            pltpu.sync_copy(gd.at[pl.ds(off, CHUNK)], dst_a.at[0])
            pltpu.async_copy(tab.at[idx_a.at[pl.ds(0, CHUNK)]],
                             rows_a.at[pl.ds(0, CHUNK)], gs_a).wait()
            pltpu.sync_copy(rows_a.at[pl.ds(0, CHUNK)],
                            acc.at[dst_a.at[0]], add=True)
            return carry

        if nch - npair * 2 * G:
            lax.fori_loop(0, nch - npair * 2 * G, rem_step, 0)

    process(taa, gaa, daa, PAD_AA, acc)
    process(tma, gma, dma, PAD_MA, acc)
    plsc.subcore_barrier()

    pltpu.sync_copy(acc.at[pl.ds(_m8(sid * 624), 624)],
                    out_a.at[pl.ds(_m8(cid * N_A + sid * 624), 624)])

    @pl.when(sid == 15)
    def _():
        pltpu.sync_copy(acc.at[pl.ds(16 * 624, N_A - 16 * 624)],
                        out_a.at[pl.ds(_m8(cid * N_A + 16 * 624),
                                       N_A - 16 * 624)])

    plsc.subcore_barrier()
    pltpu.sync_copy(zrows.at[pl.ds(0, ACC_M // 16)],
                    acc.at[pl.ds(_m8(sid * (ACC_M // 16)), ACC_M // 16)])
    plsc.subcore_barrier()

    process(tmm, gmm, dmm, PAD_MM, acc)
    process(tam, gam, dam, PAD_AM, acc)
    plsc.subcore_barrier()

    pltpu.sync_copy(acc.at[pl.ds(_m8(sid * 120), 120)],
                    out_m.at[pl.ds(_m8(cid * N_M + sid * 120), 120)])

    @pl.when(sid == 15)
    def _():
        pltpu.sync_copy(acc.at[pl.ds(16 * 120, N_M - 16 * 120)],
                        out_m.at[pl.ds(_m8(cid * N_M + 16 * 120),
                                       N_M - 16 * 120)])


def _edge_arrays(src, dst, code, ncodes, nrows_half, epad, acc_rows, nreal):
    gidx = src * ncodes + code
    e = gidx.shape[0]
    pad = epad - e
    ar = jnp.arange(pad, dtype=jnp.int32)
    gidx = jnp.concatenate([gidx, (ar * 37) % nrows_half])
    dst = jnp.concatenate([dst, nreal + ar % (acc_rows - nreal)])
    gcat = jnp.concatenate([gidx, gidx + nrows_half])
    return gcat, dst


def kernel(atom_tables, bond_aa_tables, motif_table, am_table, mm_table, ma_table,
           Wa1, ba1, Wa2, ba2, Wm1, bm1, Wm2, bm2,
           x_atom, x_motif, edge_index_aa, edge_attr_aa, edge_index_am, edge_attr_am,
           edge_index_mm, edge_attr_mm, edge_index_ma, edge_attr_ma,
           batch_atom, batch_motif):
    Da = atom_tables[:, 1, :] - atom_tables[:, 0, :]
    base_a = jnp.sum(atom_tables[:, 0, :], axis=0)[None, :]
    bits = (jnp.arange(8, dtype=jnp.int32)[:, None]
            >> jnp.arange(3, dtype=jnp.int32)[None, :]) & 1
    lut_aa = (bond_aa_tables[0][bits[:, 0]]
              + bond_aa_tables[1][bits[:, 1]]
              + bond_aa_tables[2][bits[:, 2]])
    motif_table_pad = jnp.zeros((64, D), F32).at[:61].set(motif_table)

    code_aa = (edge_attr_aa[:, 0] + 2 * edge_attr_aa[:, 1]
               + 4 * edge_attr_aa[:, 2])
    gaa, daa = _edge_arrays(edge_index_aa[0], edge_index_aa[1], code_aa,
                            8, 8 * N_A, PAD_AA, ACC_A, N_A)
    gma, dma = _edge_arrays(edge_index_ma[0], edge_index_ma[1], edge_attr_ma,
                            2, 2 * N_M, PAD_MA, ACC_A, N_A)
    gmm, dmm = _edge_arrays(edge_index_mm[0], edge_index_mm[1], edge_attr_mm,
                            22, 22 * N_M, PAD_MM, ACC_M, N_M)
    gam, dam = _edge_arrays(edge_index_am[0], edge_index_am[1], edge_attr_am,
                            2, 2 * N_M, PAD_AM, ACC_M, N_M)
    zrows = jnp.zeros((ZROWS, 128), F32)

    h_a = _encode_atoms(x_atom, Da, base_a)
    h_m = _encode_motifs(x_motif, motif_table_pad)

    xs_a = [h_a]
    xs_m = [h_m]
    for l in range(L):
        taa = _build_table(h_a, lut_aa, 1000)
        tma = _build_table(h_m, ma_table, N_M)
        tmm = _build_table(h_m, mm_table, 200)
        tam = _build_table(h_a[:N_M], am_table, N_M)
        agg_a, agg_m = _get_sc_aggregate()(taa, tma, tmm, tam,
                                           gaa, daa, gma, dma, gmm, dmm,
                                           gam, dam, zrows)
        h_a = _mlp(h_a, agg_a, Wa1[l], ba1[l][None, :], Wa2[l], ba2[l][None, :])
        h_m = _mlp(h_m, agg_m, Wm1[l], bm1[l][None, :], Wm2[l], bm2[l][None, :])
        xs_a.append(h_a)
        xs_m.append(h_m)

    atom_embs = _pool(xs_a, jnp.reshape(batch_atom, (N_A // 2000, 1, 2000)))
    motif_embs = _pool(xs_m, jnp.reshape(batch_motif, (1, 1, 2000)))
    return jnp.concatenate([atom_embs, motif_embs], axis=1)


# speedup vs baseline: 5.4886x; 1.0268x over previous
import functools

import jax
import jax.numpy as jnp
from jax import lax
from jax.experimental import pallas as pl
from jax.experimental.pallas import tpu as pltpu
from jax.experimental.pallas import tpu_sc as plsc

F32 = jnp.float32
HIGHEST = lax.Precision.HIGHEST

N_A = 10000
N_M = 2000
D = 256
L = 3
B = 64

CHUNK = 128
NBUF = 3
EDGE_ALIGN = 16 * CHUNK
ACC_A = 10048
ACC_M = 2048
ZROWS = 624


def _ceil_to(x, m):
    return ((x + m - 1) // m) * m


def _encode_atoms(x_atom, Da, base_a):
    Bn = 2000
    nb = N_A // Bn

    def body(x_ref, da_ref, base_ref, o_ref):
        xf = x_ref[...].astype(F32)
        acc = jnp.broadcast_to(base_ref[...], (Bn, D))
        for i in range(9):
            acc = acc + xf[:, i:i + 1] * da_ref[i:i + 1, :]
        o_ref[...] = acc

    return pl.pallas_call(
        body,
        grid=(nb,),
        in_specs=[
            pl.BlockSpec((Bn, 9), lambda j: (j, 0)),
            pl.BlockSpec((9, D), lambda j: (0, 0)),
            pl.BlockSpec((1, D), lambda j: (0, 0)),
        ],
        out_specs=pl.BlockSpec((Bn, D), lambda j: (j, 0)),
        out_shape=jax.ShapeDtypeStruct((N_A, D), F32),
    )(x_atom, Da, base_a)


def _encode_motifs(x_motif, motif_table_pad):
    def body(ids_ref, tab_ref, o_ref):
        ids = ids_ref[...]
        iota = lax.broadcasted_iota(jnp.int32, (1, 64), 1)
        oh = (ids == iota).astype(F32)
        o_ref[...] = jnp.dot(oh, tab_ref[...], precision=HIGHEST)

    return pl.pallas_call(
        body,
        in_specs=[
            pl.BlockSpec((N_M, 1), lambda: (0, 0)),
            pl.BlockSpec((64, D), lambda: (0, 0)),
        ],
        out_specs=pl.BlockSpec((N_M, D), lambda: (0, 0)),
        out_shape=jax.ShapeDtypeStruct((N_M, D), F32),
    )(x_motif, motif_table_pad)


def _build_table(h, lut, bv):
    n = h.shape[0]
    c = lut.shape[0]
    nb = n // bv

    def body(h_ref, lut_ref, o_ref):
        t = jnp.maximum(h_ref[...][:, None, :] + lut_ref[...][None, :, :], 0.0)
        o_ref[...] = t.reshape(bv * c, 128)

    return pl.pallas_call(
        body,
        grid=(2, nb),
        in_specs=[
            pl.BlockSpec((bv, 128), lambda hf, j: (j, hf)),
            pl.BlockSpec((c, 128), lambda hf, j: (0, hf)),
        ],
        out_specs=pl.BlockSpec((bv * c, 128), lambda hf, j: (hf * nb + j, 0)),
        out_shape=jax.ShapeDtypeStruct((2 * n * c, 128), F32),
    )(h, lut)


def _mlp(h, agg, w1, b1, w2, b2):
    n = h.shape[0]
    bn = 2000
    nb = n // bn

    def body(h_ref, a0_ref, a1_ref, w1_ref, b1_ref, w2_ref, b2_ref, o_ref):
        x = h_ref[...] + jnp.concatenate([a0_ref[...], a1_ref[...]], axis=1)
        y = jnp.maximum(jnp.dot(x, w1_ref[...], precision=HIGHEST) + b1_ref[...], 0.0)
        o_ref[...] = jnp.maximum(
            jnp.dot(y, w2_ref[...], precision=HIGHEST) + b2_ref[...], 0.0)

    return pl.pallas_call(
        body,
        grid=(nb,),
        in_specs=[
            pl.BlockSpec((bn, D), lambda j: (j, 0)),
            pl.BlockSpec((bn, 128), lambda j: (j, 0)),
            pl.BlockSpec((bn, 128), lambda j: (nb + j, 0)),
            pl.BlockSpec((D, D), lambda j: (0, 0)),
            pl.BlockSpec((1, D), lambda j: (0, 0)),
            pl.BlockSpec((D, D), lambda j: (0, 0)),
            pl.BlockSpec((1, D), lambda j: (0, 0)),
        ],
        out_specs=pl.BlockSpec((bn, D), lambda j: (j, 0)),
        out_shape=jax.ShapeDtypeStruct((n, D), F32),
    )(h, agg, agg, w1, b1, w2, b2)


def _pool(xs, batch3d):
    n = xs[0].shape[0]
    bn = 2000
    nb = n // bn

    def body(b_ref, x0, x1, x2, x3, o_ref):
        j = pl.program_id(0)
        ids = b_ref[0, 0, :]
        iota = lax.broadcasted_iota(jnp.int32, (B, bn), 0)
        oh = (iota == ids[None, :]).astype(F32)
        xcat = jnp.concatenate([x0[...], x1[...], x2[...], x3[...]], axis=1)
        part = jnp.dot(oh, xcat, precision=HIGHEST)

        @pl.when(j == 0)
        def _():
            o_ref[...] = jnp.zeros_like(o_ref)

        o_ref[...] += part

    xspec = pl.BlockSpec((bn, D), lambda j: (j, 0))
    return pl.pallas_call(
        body,
        grid=(nb,),
        in_specs=[pl.BlockSpec((1, 1, bn), lambda j: (j, 0, 0))] + [xspec] * 4,
        out_specs=pl.BlockSpec((B, 4 * D), lambda j: (0, 0)),
        out_shape=jax.ShapeDtypeStruct((B, 4 * D), F32),
    )(batch3d, *xs)


PAD_AA = _ceil_to(160000, EDGE_ALIGN)
PAD_MA = _ceil_to(20000, EDGE_ALIGN)
PAD_MM = _ceil_to(8000, EDGE_ALIGN)
PAD_AM = _ceil_to(20000, EDGE_ALIGN)

@functools.lru_cache(maxsize=1)
def _get_sc_aggregate():
    mesh = plsc.VectorSubcoreMesh(core_axis_name="c", subcore_axis_name="s")

    @functools.partial(
        pl.kernel,
        mesh=mesh,
        out_type=[
            jax.ShapeDtypeStruct((2 * N_A, 128), F32),
            jax.ShapeDtypeStruct((2 * N_M, 128), F32),
        ],
        scratch_types=(
            [pltpu.VMEM((CHUNK,), jnp.int32)] * NBUF
            + [pltpu.VMEM((1, CHUNK), jnp.int32)] * NBUF
            + [pltpu.VMEM((CHUNK, 128), F32)] * NBUF
            + [pltpu.VMEM_SHARED((ACC_A, 128), F32)]
            + [pltpu.SemaphoreType.DMA] * (3 * NBUF)
        ),
    )
    def _sc_aggregate(taa, tma, tmm, tam, gaa, daa, gma, dma, gmm, dmm,
                      gam, dam, zrows, out_a, out_m, *bufs):
        _sc_body(taa, tma, tmm, tam, gaa, daa, gma, dma, gmm, dmm, gam, dam,
                 zrows, out_a, out_m, bufs)

    return _sc_aggregate


def _m8(x):
    return pl.multiple_of(x, 8)


def _sc_body(taa, tma, tmm, tam, gaa, daa, gma, dma, gmm, dmm, gam, dam,
             zrows, out_a, out_m, bufs):
    idxs = bufs[0:NBUF]
    dsts = bufs[NBUF:2 * NBUF]
    rows = bufs[2 * NBUF:3 * NBUF]
    acc = bufs[3 * NBUF]
    isems = bufs[3 * NBUF + 1:3 * NBUF + 1 + NBUF]
    gsems = bufs[3 * NBUF + 1 + NBUF:3 * NBUF + 1 + 2 * NBUF]
    ssems = bufs[3 * NBUF + 1 + 2 * NBUF:3 * NBUF + 1 + 3 * NBUF]
    cid = lax.axis_index("c")
    sid = lax.axis_index("s")

    pltpu.sync_copy(zrows, acc.at[pl.ds(_m8(sid * 624), 624)])

    @pl.when(sid == 15)
    def _():
        pltpu.sync_copy(zrows.at[pl.ds(0, ACC_A - 16 * 624)],
                        acc.at[pl.ds(16 * 624, ACC_A - 16 * 624)])

    plsc.subcore_barrier()

    def process(tab, gcat, gd, epad):
        per = epad // 16
        nch = per // CHUNK
        nround = nch // NBUF
        base = sid * per

        def load(c, b):
            off = _m8(base + c * CHUNK)
            return [
                pltpu.async_copy(
                    gcat.at[pl.ds(_m8(cid * epad + off), CHUNK)],
                    idxs[b], isems[b]),
                pltpu.async_copy(gd.at[pl.ds(off, CHUNK)], dsts[b].at[0],
                                 isems[b]),
            ]

        def gather(b):
            return pltpu.async_copy(tab.at[idxs[b]], rows[b], gsems[b])

        def scatter(b):
            return pltpu.async_copy(rows[b], acc.at[dsts[b].at[0]],
                                    ssems[b], add=True)

        def round_(i, carry):
            c0 = i * NBUF
            ls = [load(c0 + b, b) for b in range(NBUF)]
            gs = []
            for b in range(NBUF):
                for d in ls[b]:
                    d.wait()
                gs.append(gather(b))
            ss = []
            for b in range(NBUF):
                gs[b].wait()
                ss.append(scatter(b))
            for d in ss:
                d.wait()
            return carry

        lax.fori_loop(0, nround, round_, 0)

        def rem_step(i, carry):
            c = nround * NBUF + i
            off = _m8(base + c * CHUNK)
            pltpu.sync_copy(gcat.at[pl.ds(_m8(cid * epad + off), CHUNK)],
                            idxs[0])
            pltpu.sync_copy(gd.at[pl.ds(off, CHUNK)], dsts[0].at[0])
            gather(0).wait()
            scatter(0).wait()
            return carry

        if nch - (nch // NBUF) * NBUF:
            lax.fori_loop(0, nch - nround * NBUF, rem_step, 0)

    process(taa, gaa, daa, PAD_AA)
    process(tma, gma, dma, PAD_MA)
    plsc.subcore_barrier()

    pltpu.sync_copy(acc.at[pl.ds(_m8(sid * 624), 624)],
                    out_a.at[pl.ds(_m8(cid * N_A + sid * 624), 624)])

    @pl.when(sid == 15)
    def _():
        pltpu.sync_copy(acc.at[pl.ds(16 * 624, N_A - 16 * 624)],
                        out_a.at[pl.ds(_m8(cid * N_A + 16 * 624),
                                       N_A - 16 * 624)])

    plsc.subcore_barrier()
    pltpu.sync_copy(zrows.at[pl.ds(0, ACC_M // 16)],
                    acc.at[pl.ds(_m8(sid * (ACC_M // 16)), ACC_M // 16)])
    plsc.subcore_barrier()

    process(tmm, gmm, dmm, PAD_MM)
    process(tam, gam, dam, PAD_AM)
    plsc.subcore_barrier()

    pltpu.sync_copy(acc.at[pl.ds(_m8(sid * 120), 120)],
                    out_m.at[pl.ds(_m8(cid * N_M + sid * 120), 120)])

    @pl.when(sid == 15)
    def _():
        pltpu.sync_copy(acc.at[pl.ds(16 * 120, N_M - 16 * 120)],
                        out_m.at[pl.ds(_m8(cid * N_M + 16 * 120),
                                       N_M - 16 * 120)])


def _edge_arrays(src, dst, code, ncodes, nrows_half, epad, acc_rows, nreal):
    gidx = src * ncodes + code
    e = gidx.shape[0]
    pad = epad - e
    ar = jnp.arange(pad, dtype=jnp.int32)
    gidx = jnp.concatenate([gidx, (ar * 37) % nrows_half])
    dst = jnp.concatenate([dst, nreal + ar % (acc_rows - nreal)])
    gcat = jnp.concatenate([gidx, gidx + nrows_half])
    return gcat, dst


def kernel(atom_tables, bond_aa_tables, motif_table, am_table, mm_table, ma_table,
           Wa1, ba1, Wa2, ba2, Wm1, bm1, Wm2, bm2,
           x_atom, x_motif, edge_index_aa, edge_attr_aa, edge_index_am, edge_attr_am,
           edge_index_mm, edge_attr_mm, edge_index_ma, edge_attr_ma,
           batch_atom, batch_motif):
    Da = atom_tables[:, 1, :] - atom_tables[:, 0, :]
    base_a = jnp.sum(atom_tables[:, 0, :], axis=0)[None, :]
    bits = (jnp.arange(8, dtype=jnp.int32)[:, None]
            >> jnp.arange(3, dtype=jnp.int32)[None, :]) & 1
    lut_aa = (bond_aa_tables[0][bits[:, 0]]
              + bond_aa_tables[1][bits[:, 1]]
              + bond_aa_tables[2][bits[:, 2]])
    motif_table_pad = jnp.zeros((64, D), F32).at[:61].set(motif_table)

    code_aa = (edge_attr_aa[:, 0] + 2 * edge_attr_aa[:, 1]
               + 4 * edge_attr_aa[:, 2])
    gaa, daa = _edge_arrays(edge_index_aa[0], edge_index_aa[1], code_aa,
                            8, 8 * N_A, PAD_AA, ACC_A, N_A)
    gma, dma = _edge_arrays(edge_index_ma[0], edge_index_ma[1], edge_attr_ma,
                            2, 2 * N_M, PAD_MA, ACC_A, N_A)
    gmm, dmm = _edge_arrays(edge_index_mm[0], edge_index_mm[1], edge_attr_mm,
                            22, 22 * N_M, PAD_MM, ACC_M, N_M)
    gam, dam = _edge_arrays(edge_index_am[0], edge_index_am[1], edge_attr_am,
                            2, 2 * N_M, PAD_AM, ACC_M, N_M)
    zrows = jnp.zeros((ZROWS, 128), F32)

    h_a = _encode_atoms(x_atom, Da, base_a)
    h_m = _encode_motifs(x_motif, motif_table_pad)

    xs_a = [h_a]
    xs_m = [h_m]
    for l in range(L):
        taa = _build_table(h_a, lut_aa, 1000)
        tma = _build_table(h_m, ma_table, N_M)
        tmm = _build_table(h_m, mm_table, 200)
        tam = _build_table(h_a[:N_M], am_table, N_M)
        agg_a, agg_m = _get_sc_aggregate()(taa, tma, tmm, tam,
                                           gaa, daa, gma, dma, gmm, dmm,
                                           gam, dam, zrows)
        h_a = _mlp(h_a, agg_a, Wa1[l], ba1[l][None, :], Wa2[l], ba2[l][None, :])
        h_m = _mlp(h_m, agg_m, Wm1[l], bm1[l][None, :], Wm2[l], bm2[l][None, :])
        xs_a.append(h_a)
        xs_m.append(h_m)

    atom_embs = _pool(xs_a, jnp.reshape(batch_atom, (N_A // 2000, 1, 2000)))
    motif_embs = _pool(xs_m, jnp.reshape(batch_motif, (1, 1, 2000)))
    return jnp.concatenate([atom_embs, motif_embs], axis=1)


# speedup vs baseline: 5.6282x; 1.0254x over previous
import functools

import jax
import jax.numpy as jnp
from jax import lax
from jax.experimental import pallas as pl
from jax.experimental.pallas import tpu as pltpu
from jax.experimental.pallas import tpu_sc as plsc

F32 = jnp.float32
HIGHEST = lax.Precision.HIGHEST

N_A = 10000
N_M = 2000
D = 256
L = 3
B = 64

CHUNK = 128
NBUF = 3
EDGE_ALIGN = 16 * CHUNK
ACC_A = 10048
ACC_M = 2048
ZROWS = 624


def _ceil_to(x, m):
    return ((x + m - 1) // m) * m


def _encode_atoms(x_atom, Da, base_a):
    Bn = 2000
    nb = N_A // Bn

    def body(x_ref, da_ref, base_ref, o_ref):
        xf = x_ref[...].astype(F32)
        acc = jnp.broadcast_to(base_ref[...], (Bn, D))
        for i in range(9):
            acc = acc + xf[:, i:i + 1] * da_ref[i:i + 1, :]
        o_ref[...] = acc

    return pl.pallas_call(
        body,
        grid=(nb,),
        in_specs=[
            pl.BlockSpec((Bn, 9), lambda j: (j, 0)),
            pl.BlockSpec((9, D), lambda j: (0, 0)),
            pl.BlockSpec((1, D), lambda j: (0, 0)),
        ],
        out_specs=pl.BlockSpec((Bn, D), lambda j: (j, 0)),
        out_shape=jax.ShapeDtypeStruct((N_A, D), F32),
    )(x_atom, Da, base_a)


def _encode_motifs(x_motif, motif_table_pad):
    def body(ids_ref, tab_ref, o_ref):
        ids = ids_ref[...]
        iota = lax.broadcasted_iota(jnp.int32, (1, 64), 1)
        oh = (ids == iota).astype(F32)
        o_ref[...] = jnp.dot(oh, tab_ref[...], precision=HIGHEST)

    return pl.pallas_call(
        body,
        in_specs=[
            pl.BlockSpec((N_M, 1), lambda: (0, 0)),
            pl.BlockSpec((64, D), lambda: (0, 0)),
        ],
        out_specs=pl.BlockSpec((N_M, D), lambda: (0, 0)),
        out_shape=jax.ShapeDtypeStruct((N_M, D), F32),
    )(x_motif, motif_table_pad)


def _build_tables2(h, lut1, lut2, bv):
    n = h.shape[0]
    c1 = lut1.shape[0]
    c2 = lut2.shape[0]
    nb = n // bv

    def body(h_ref, l1_ref, l2_ref, o1_ref, o2_ref):
        hv = h_ref[...]
        t1 = jnp.maximum(hv[:, None, :] + l1_ref[...][None, :, :], 0.0)
        o1_ref[...] = t1.reshape(bv * c1, 128)
        t2 = jnp.maximum(hv[:, None, :] + l2_ref[...][None, :, :], 0.0)
        o2_ref[...] = t2.reshape(bv * c2, 128)

    return pl.pallas_call(
        body,
        grid=(2, nb),
        in_specs=[
            pl.BlockSpec((bv, 128), lambda hf, j: (j, hf)),
            pl.BlockSpec((c1, 128), lambda hf, j: (0, hf)),
            pl.BlockSpec((c2, 128), lambda hf, j: (0, hf)),
        ],
        out_specs=[
            pl.BlockSpec((bv * c1, 128), lambda hf, j: (hf * nb + j, 0)),
            pl.BlockSpec((bv * c2, 128), lambda hf, j: (hf * nb + j, 0)),
        ],
        out_shape=[
            jax.ShapeDtypeStruct((2 * n * c1, 128), F32),
            jax.ShapeDtypeStruct((2 * n * c2, 128), F32),
        ],
    )(h, lut1, lut2)


def _mlp(h, agg, w1, b1, w2, b2):
    n = h.shape[0]
    bn = 2000
    nb = n // bn

    def body(h_ref, a0_ref, a1_ref, w1_ref, b1_ref, w2_ref, b2_ref, o_ref):
        x = h_ref[...] + jnp.concatenate([a0_ref[...], a1_ref[...]], axis=1)
        y = jnp.maximum(jnp.dot(x, w1_ref[...], precision=HIGHEST) + b1_ref[...], 0.0)
        o_ref[...] = jnp.maximum(
            jnp.dot(y, w2_ref[...], precision=HIGHEST) + b2_ref[...], 0.0)

    return pl.pallas_call(
        body,
        grid=(nb,),
        in_specs=[
            pl.BlockSpec((bn, D), lambda j: (j, 0)),
            pl.BlockSpec((bn, 128), lambda j: (j, 0)),
            pl.BlockSpec((bn, 128), lambda j: (nb + j, 0)),
            pl.BlockSpec((D, D), lambda j: (0, 0)),
            pl.BlockSpec((1, D), lambda j: (0, 0)),
            pl.BlockSpec((D, D), lambda j: (0, 0)),
            pl.BlockSpec((1, D), lambda j: (0, 0)),
        ],
        out_specs=pl.BlockSpec((bn, D), lambda j: (j, 0)),
        out_shape=jax.ShapeDtypeStruct((n, D), F32),
    )(h, agg, agg, w1, b1, w2, b2)


def _pool(xs, batch3d):
    n = xs[0].shape[0]
    bn = 2000
    nb = n // bn

    def body(b_ref, x0, x1, x2, x3, o_ref):
        j = pl.program_id(0)
        ids = b_ref[0, 0, :]
        iota = lax.broadcasted_iota(jnp.int32, (B, bn), 0)
        oh = (iota == ids[None, :]).astype(F32)
        xcat = jnp.concatenate([x0[...], x1[...], x2[...], x3[...]], axis=1)
        part = jnp.dot(oh, xcat, precision=HIGHEST)

        @pl.when(j == 0)
        def _():
            o_ref[...] = jnp.zeros_like(o_ref)

        o_ref[...] += part

    xspec = pl.BlockSpec((bn, D), lambda j: (j, 0))
    return pl.pallas_call(
        body,
        grid=(nb,),
        in_specs=[pl.BlockSpec((1, 1, bn), lambda j: (j, 0, 0))] + [xspec] * 4,
        out_specs=pl.BlockSpec((B, 4 * D), lambda j: (0, 0)),
        out_shape=jax.ShapeDtypeStruct((B, 4 * D), F32),
    )(batch3d, *xs)


PAD_AA = _ceil_to(160000, EDGE_ALIGN)
PAD_MA = _ceil_to(20000, EDGE_ALIGN)
PAD_MM = _ceil_to(8000, EDGE_ALIGN)
PAD_AM = _ceil_to(20000, EDGE_ALIGN)

@functools.lru_cache(maxsize=1)
def _get_sc_aggregate():
    mesh = plsc.VectorSubcoreMesh(core_axis_name="c", subcore_axis_name="s")

    @functools.partial(
        pl.kernel,
        mesh=mesh,
        out_type=[
            jax.ShapeDtypeStruct((2 * N_A, 128), F32),
            jax.ShapeDtypeStruct((2 * N_M, 128), F32),
        ],
        scratch_types=(
            [pltpu.VMEM((CHUNK,), jnp.int32)] * NBUF
            + [pltpu.VMEM((1, CHUNK), jnp.int32)] * NBUF
            + [pltpu.VMEM((CHUNK, 128), F32)] * NBUF
            + [pltpu.VMEM_SHARED((ACC_A, 128), F32)]
            + [pltpu.SemaphoreType.DMA] * (3 * NBUF)
        ),
    )
    def _sc_aggregate(taa, tma, tmm, tam, gaa, daa, gma, dma, gmm, dmm,
                      gam, dam, zrows, out_a, out_m, *bufs):
        _sc_body(taa, tma, tmm, tam, gaa, daa, gma, dma, gmm, dmm, gam, dam,
                 zrows, out_a, out_m, bufs)

    return _sc_aggregate


def _m8(x):
    return pl.multiple_of(x, 8)


def _sc_body(taa, tma, tmm, tam, gaa, daa, gma, dma, gmm, dmm, gam, dam,
             zrows, out_a, out_m, bufs):
    idxs = bufs[0:NBUF]
    dsts = bufs[NBUF:2 * NBUF]
    rows = bufs[2 * NBUF:3 * NBUF]
    acc = bufs[3 * NBUF]
    isems = bufs[3 * NBUF + 1:3 * NBUF + 1 + NBUF]
    gsems = bufs[3 * NBUF + 1 + NBUF:3 * NBUF + 1 + 2 * NBUF]
    ssems = bufs[3 * NBUF + 1 + 2 * NBUF:3 * NBUF + 1 + 3 * NBUF]
    cid = lax.axis_index("c")
    sid = lax.axis_index("s")

    pltpu.sync_copy(zrows, acc.at[pl.ds(_m8(sid * 624), 624)])

    @pl.when(sid == 15)
    def _():
        pltpu.sync_copy(zrows.at[pl.ds(0, ACC_A - 16 * 624)],
                        acc.at[pl.ds(16 * 624, ACC_A - 16 * 624)])

    plsc.subcore_barrier()

    def process(tab, gcat, gd, epad):
        per = epad // 16
        nch = per // CHUNK
        nround = nch // NBUF
        base = sid * per

        def load(c, b):
            off = _m8(base + c * CHUNK)
            return [
                pltpu.async_copy(
                    gcat.at[pl.ds(_m8(cid * epad + off), CHUNK)],
                    idxs[b], isems[b]),
                pltpu.async_copy(gd.at[pl.ds(off, CHUNK)], dsts[b].at[0],
                                 isems[b]),
            ]

        def gather(b):
            return pltpu.async_copy(tab.at[idxs[b]], rows[b], gsems[b])

        def scatter(b):
            return pltpu.async_copy(rows[b], acc.at[dsts[b].at[0]],
                                    ssems[b], add=True)

        def round_(i, carry):
            c0 = i * NBUF
            ls = [load(c0 + b, b) for b in range(NBUF)]
            gs = []
            for b in range(NBUF):
                for d in ls[b]:
                    d.wait()
                gs.append(gather(b))
            ss = []
            for b in range(NBUF):
                gs[b].wait()
                ss.append(scatter(b))
            for d in ss:
                d.wait()
            return carry

        lax.fori_loop(0, nround, round_, 0)

        def rem_step(i, carry):
            c = nround * NBUF + i
            off = _m8(base + c * CHUNK)
            pltpu.sync_copy(gcat.at[pl.ds(_m8(cid * epad + off), CHUNK)],
                            idxs[0])
            pltpu.sync_copy(gd.at[pl.ds(off, CHUNK)], dsts[0].at[0])
            gather(0).wait()
            scatter(0).wait()
            return carry

        if nch - (nch // NBUF) * NBUF:
            lax.fori_loop(0, nch - nround * NBUF, rem_step, 0)

    process(taa, gaa, daa, PAD_AA)
    process(tma, gma, dma, PAD_MA)
    plsc.subcore_barrier()

    pltpu.sync_copy(acc.at[pl.ds(_m8(sid * 624), 624)],
                    out_a.at[pl.ds(_m8(cid * N_A + sid * 624), 624)])

    @pl.when(sid == 15)
    def _():
        pltpu.sync_copy(acc.at[pl.ds(16 * 624, N_A - 16 * 624)],
                        out_a.at[pl.ds(_m8(cid * N_A + 16 * 624),
                                       N_A - 16 * 624)])

    plsc.subcore_barrier()
    pltpu.sync_copy(zrows.at[pl.ds(0, ACC_M // 16)],
                    acc.at[pl.ds(_m8(sid * (ACC_M // 16)), ACC_M // 16)])
    plsc.subcore_barrier()

    process(tmm, gmm, dmm, PAD_MM)
    process(tam, gam, dam, PAD_AM)
    plsc.subcore_barrier()

    pltpu.sync_copy(acc.at[pl.ds(_m8(sid * 120), 120)],
                    out_m.at[pl.ds(_m8(cid * N_M + sid * 120), 120)])

    @pl.when(sid == 15)
    def _():
        pltpu.sync_copy(acc.at[pl.ds(16 * 120, N_M - 16 * 120)],
                        out_m.at[pl.ds(_m8(cid * N_M + 16 * 120),
                                       N_M - 16 * 120)])


def _edge_arrays(src, dst, code, ncodes, nrows_half, epad, acc_rows, nreal):
    gidx = src * ncodes + code
    e = gidx.shape[0]
    pad = epad - e
    ar = jnp.arange(pad, dtype=jnp.int32)
    gidx = jnp.concatenate([gidx, (ar * 37) % nrows_half])
    dst = jnp.concatenate([dst, nreal + ar % (acc_rows - nreal)])
    gcat = jnp.concatenate([gidx, gidx + nrows_half])
    return gcat, dst


def kernel(atom_tables, bond_aa_tables, motif_table, am_table, mm_table, ma_table,
           Wa1, ba1, Wa2, ba2, Wm1, bm1, Wm2, bm2,
           x_atom, x_motif, edge_index_aa, edge_attr_aa, edge_index_am, edge_attr_am,
           edge_index_mm, edge_attr_mm, edge_index_ma, edge_attr_ma,
           batch_atom, batch_motif):
    Da = atom_tables[:, 1, :] - atom_tables[:, 0, :]
    base_a = jnp.sum(atom_tables[:, 0, :], axis=0)[None, :]
    bits = (jnp.arange(8, dtype=jnp.int32)[:, None]
            >> jnp.arange(3, dtype=jnp.int32)[None, :]) & 1
    lut_aa = (bond_aa_tables[0][bits[:, 0]]
              + bond_aa_tables[1][bits[:, 1]]
              + bond_aa_tables[2][bits[:, 2]])
    motif_table_pad = jnp.zeros((64, D), F32).at[:61].set(motif_table)

    code_aa = (edge_attr_aa[:, 0] + 2 * edge_attr_aa[:, 1]
               + 4 * edge_attr_aa[:, 2])
    gaa, daa = _edge_arrays(edge_index_aa[0], edge_index_aa[1], code_aa,
                            8, 8 * N_A, PAD_AA, ACC_A, N_A)
    gma, dma = _edge_arrays(edge_index_ma[0], edge_index_ma[1], edge_attr_ma,
                            2, 2 * N_M, PAD_MA, ACC_A, N_A)
    gmm, dmm = _edge_arrays(edge_index_mm[0], edge_index_mm[1], edge_attr_mm,
                            22, 22 * N_M, PAD_MM, ACC_M, N_M)
    gam, dam = _edge_arrays(edge_index_am[0], edge_index_am[1], edge_attr_am,
                            2, 2 * N_A, PAD_AM, ACC_M, N_M)
    zrows = jnp.zeros((ZROWS, 128), F32)

    h_a = _encode_atoms(x_atom, Da, base_a)
    h_m = _encode_motifs(x_motif, motif_table_pad)

    xs_a = [h_a]
    xs_m = [h_m]
    for l in range(L):
        taa, tam = _build_tables2(h_a, lut_aa, am_table, 1000)
        tma, tmm = _build_tables2(h_m, ma_table, mm_table, 200)
        agg_a, agg_m = _get_sc_aggregate()(taa, tma, tmm, tam,
                                           gaa, daa, gma, dma, gmm, dmm,
                                           gam, dam, zrows)
        h_a = _mlp(h_a, agg_a, Wa1[l], ba1[l][None, :], Wa2[l], ba2[l][None, :])
        h_m = _mlp(h_m, agg_m, Wm1[l], bm1[l][None, :], Wm2[l], bm2[l][None, :])
        xs_a.append(h_a)
        xs_m.append(h_m)

    atom_embs = _pool(xs_a, jnp.reshape(batch_atom, (N_A // 2000, 1, 2000)))
    motif_embs = _pool(xs_m, jnp.reshape(batch_motif, (1, 1, 2000)))
    return jnp.concatenate([atom_embs, motif_embs], axis=1)
